# baseline (speedup 1.0000x reference)
import functools

import jax
import jax.numpy as jnp
from jax import lax
from jax.experimental import pallas as pl
from jax.experimental.pallas import tpu as pltpu, tpu_sc as plsc

_NC, _NS, _L = 2, 16, 16
_NW = _NC * _NS
_U, _F = 1024, 2048
_UPW = _U // _NW
_G = 8
_NG = _UPW // _G
_NV = _F // _L
_CAND = 256
_CAP = 192
_GU = 8
_NEG = jnp.float32(-3.0e38)
_RT = jnp.float32(12582912.0)

_mesh = plsc.VectorSubcoreMesh(core_axis_name="c", subcore_axis_name="s")
_sc_params = pltpu.CompilerParams(needs_layout_passes=False)


def _scalar(vec, i):
    return lax.squeeze(lax.slice(vec, (i,), (i + 1,)), (0,))


def _topk_kernel(d_hbm, gn_hbm, w_hbm, idx_hbm, w8_hbm, wmax_hbm,
                 db, gb, wb, cv, ci, idxst, w8st, wmaxst, sems):
    wid = lax.axis_index("s") * _NC + lax.axis_index("c")
    ubase = wid * _UPW
    lane = lax.iota(jnp.int32, _L)

    def start_group(g, slot):
        rows = ubase + g * _G
        pltpu.async_copy(d_hbm.at[pl.ds(rows, _G)], db.at[slot], sems.at[slot, 0])
        pltpu.async_copy(gn_hbm.at[pl.ds(rows, _G)], gb.at[slot], sems.at[slot, 1])
        pltpu.async_copy(w_hbm.at[pl.ds(rows, _G)], wb.at[slot], sems.at[slot, 2])

    def wait_group(g, slot):
        rows = ubase + g * _G
        pltpu.make_async_copy(d_hbm.at[pl.ds(rows, _G)], db.at[slot],
                              sems.at[slot, 0]).wait()
        pltpu.make_async_copy(gn_hbm.at[pl.ds(rows, _G)], gb.at[slot],
                              sems.at[slot, 1]).wait()
        pltpu.make_async_copy(w_hbm.at[pl.ds(rows, _G)], wb.at[slot],
                              sems.at[slot, 2]).wait()

    start_group(0, 0)

    def do_group(g, wmax_acc):
        slot = lax.rem(g, 2)

        @pl.when(g + 1 < _NG)
        def _():
            start_group(g + 1, lax.rem(g + 1, 2))

        wait_group(g, slot)

        def do_unit(u, wmax_acc):
            def pa(i, acc):
                sl = pl.ds(i * _L, _L)
                p = db[slot, u, sl] + gb[slot, u, sl]
                db[slot, u, sl] = p
                return jnp.maximum(acc, p)
            acc = lax.fori_loop(0, _NV, pa, jnp.full((_L,), _NEG, jnp.float32))
            t = jnp.max(acc)
            for _ in range(7):
                acc = jnp.where(acc == t, _NEG, acc)
                t = jnp.max(acc)
            tv = jnp.full((_L,), t, jnp.float32)

            def pf(i, _):
                sl = pl.ds(i * _L, _L)
                cv[sl] = jnp.full((_L,), _NEG, jnp.float32)
                ci[sl] = jnp.full((_L,), 2047, jnp.int32)
                return 0
            lax.fori_loop(0, _CAND // _L, pf, 0)

            def ex(i, off):
                sl = pl.ds(i * _L, _L)
                p = db[slot, u, sl]
                msk = p >= tv
                plsc.store_compressed(cv.at[pl.ds(off, _L)], p, mask=msk)
                plsc.store_compressed(ci.at[pl.ds(off, _L)], lane + i * _L,
                                      mask=msk)
                cnt = _scalar(plsc.all_reduce_population_count(msk), 0)
                return jnp.minimum(off + cnt, _CAP)
            lax.fori_loop(0, _NV, ex, jnp.int32(0))

            res_i = jnp.zeros((_L,), jnp.int32)
            for r in range(8):
                bv = cv[pl.ds(0, _L)]
                bi = ci[pl.ds(0, _L)]
                for j in range(1, 4):
                    v = cv[pl.ds(j * _L, _L)]
                    ii = ci[pl.ds(j * _L, _L)]
                    take = (v > bv) | ((v == bv) & (ii < bi))
                    bv = jnp.where(take, v, bv)
                    bi = jnp.where(take, ii, bi)
                m = _scalar(plsc.cummax(bv), _L - 1)
                mv = jnp.full((_L,), m, jnp.float32)
                iw = jnp.where(bv == mv, 2047 - bi, 0)
                win = 2047 - _scalar(plsc.cummax(iw), _L - 1)
                winv = jnp.full((_L,), win, jnp.int32)
                res_i = jnp.where(lane == r, winv, res_i)
                for j in range(4):
                    sl = pl.ds(j * _L, _L)
                    hit = ci[sl] == winv
                    cv[sl] = jnp.where(hit, _NEG, cv[sl])
            wvals = plsc.load_gather(
                wb, [jnp.full((_L,), slot, jnp.int32),
                     jnp.full((_L,), u, jnp.int32), res_i])
            wvals = jnp.where(lane < 8, wvals, 0.0)
            idxst[u] = jnp.where(lane < 8, res_i, lax.rev(res_i, (0,)))
            w8st[u] = wvals
            return jnp.maximum(wmax_acc, jnp.abs(wvals))

        wmax_acc = lax.fori_loop(0, _G, do_unit, wmax_acc)
        rows = ubase + g * _G
        pltpu.sync_copy(idxst, idx_hbm.at[pl.ds(rows, _G)])
        pltpu.sync_copy(w8st, w8_hbm.at[pl.ds(rows, _G)])
        return wmax_acc

    wmax_acc = lax.fori_loop(0, _NG, do_group, jnp.zeros((_L,), jnp.float32))
    wmaxst[...] = wmax_acc
    pltpu.sync_copy(wmaxst, wmax_hbm.at[wid])


_topk1 = functools.partial(
    pl.kernel, mesh=_mesh, compiler_params=_sc_params,
    out_type=[
        jax.ShapeDtypeStruct((_U, _L), jnp.int32),
        jax.ShapeDtypeStruct((_U, _L), jnp.float32),
        jax.ShapeDtypeStruct((_NW, _L), jnp.float32),
    ],
    scratch_types=[
        pltpu.VMEM((2, _G, _F), jnp.float32),
        pltpu.VMEM((2, _G, _F), jnp.float32),
        pltpu.VMEM((2, _G, _F), jnp.float32),
        pltpu.VMEM((_CAND + _L,), jnp.float32),
        pltpu.VMEM((_CAND + _L,), jnp.int32),
        pltpu.VMEM((_G, _L), jnp.int32),
        pltpu.VMEM((_G, _L), jnp.float32),
        pltpu.VMEM((_L,), jnp.float32),
        pltpu.SemaphoreType.DMA((2, 3)),
    ],
)(_topk_kernel)


def _matmul_kernel(xt_hbm, idxf_hbm, w8_hbm, b_hbm, scl_hbm,
                   yt_hbm, ymax_hbm,
                   ig0, ig1, ig2, ig3, w8b, bb, scl, xg, yst, ymaxst, sems):
    wid = lax.axis_index("s") * _NC + lax.axis_index("c")
    ubase = wid * _UPW
    lane = lax.iota(jnp.int32, _L)
    ngm = _UPW // _GU
    igs = [ig0, ig1, ig2, ig3]

    for g in range(ngm):
        pltpu.sync_copy(
            idxf_hbm.at[pl.ds(wid * _UPW * _L + g * _GU * _L, _GU * _L)],
            igs[g])
    pltpu.sync_copy(w8_hbm.at[pl.ds(ubase, _UPW)], w8b)
    pltpu.sync_copy(b_hbm.at[pl.ds(ubase, _UPW)], bb)
    pltpu.sync_copy(scl_hbm, scl)
    scl_v = scl[...]

    def start_group(g):
        pltpu.async_copy(xt_hbm.at[igs[g]], xg.at[g % 2], sems.at[g % 2])

    def wait_group(g):
        pltpu.make_async_copy(xt_hbm.at[igs[g]], xg.at[g % 2],
                              sems.at[g % 2]).wait()

    start_group(0)
    start_group(1)

    ymax_acc = jnp.zeros((_L,), jnp.float32)
    for g in range(ngm):
        slot = g % 2
        wait_group(g)
        bq16 = (scl_v * bb[pl.ds((g // 2) * _L, _L)] + _RT) - _RT

        def do_unit(u, ymax_acc, g=g, slot=slot, bq16=bq16):
            gu = g * _GU + u
            wq = (scl_v * w8b[gu] + _RT) - _RT
            bqs = jnp.max(jnp.where(lane == (g % 2) * _GU + u, bq16, _NEG))
            bqv = jnp.full((_L,), bqs, jnp.float32)
            ws = [jnp.full((_L,), _scalar(wq, sidx), jnp.float32)
                  for sidx in range(8)]
            for j in range(8):
                accj = bqv
                for sidx in range(8):
                    xv = xg[slot, u * _L + sidx, pl.ds(j * _L, _L)]
                    accj = accj + ws[sidx] * xv
                yj = jnp.maximum(accj, 0.0)
                yst[gu, pl.ds(j * _L, _L)] = yj
                ymax_acc = jnp.maximum(ymax_acc, yj)
            return ymax_acc

        ymax_acc = lax.fori_loop(0, _GU, do_unit, ymax_acc)
        if g + 2 < ngm:
            start_group(g + 2)

    pltpu.sync_copy(yst, yt_hbm.at[pl.ds(ubase, _UPW)])
    ymaxst[...] = ymax_acc
    pltpu.sync_copy(ymaxst, ymax_hbm.at[wid])


_spmm2 = functools.partial(
    pl.kernel, mesh=_mesh, compiler_params=_sc_params,
    out_type=[
        jax.ShapeDtypeStruct((_U, 128), jnp.float32),
        jax.ShapeDtypeStruct((_NW, _L), jnp.float32),
    ],
    scratch_types=[
        pltpu.VMEM((_GU * _L,), jnp.int32),
        pltpu.VMEM((_GU * _L,), jnp.int32),
        pltpu.VMEM((_GU * _L,), jnp.int32),
        pltpu.VMEM((_GU * _L,), jnp.int32),
        pltpu.VMEM((_UPW, _L), jnp.float32),
        pltpu.VMEM((_UPW,), jnp.float32),
        pltpu.VMEM((_L,), jnp.float32),
        pltpu.VMEM((2, _GU * _L, 128), jnp.float32),
        pltpu.VMEM((_UPW, 128), jnp.float32),
        pltpu.VMEM((_L,), jnp.float32),
        pltpu.SemaphoreType.DMA((2,)),
    ],
)(_matmul_kernel)


def _quant_kernel(scale2_ref, y_ref, out_ref):
    out_ref[...] = jnp.round(scale2_ref[0, 0] * y_ref[...])


def kernel(x, W, b, D):
    units, feat = D.shape
    batch = x.shape[0]

    with jax.ensure_compile_time_eval():
        u_rand = jax.random.uniform(jax.random.key(1234), (1, units, feat),
                                    minval=0.0, maxval=1.0)
        gn = (-0.1 * jnp.log(-jnp.log(u_rand + 1e-20) + 1e-20))[0]

    idx, w8, wmax = _topk1(D, gn, W)
    scale = 127.5 / jnp.max(wmax)
    xt = jnp.asarray(x.T)
    yT, ymax = _spmm2(xt, jnp.reshape(idx, (units * _L,)), w8, b,
                      jnp.full((_L,), scale, jnp.float32))
    scale2 = jnp.reshape(127.5 / jnp.max(ymax), (1, 1))

    outT = pl.pallas_call(
        _quant_kernel,
        in_specs=[
            pl.BlockSpec(memory_space=pltpu.SMEM),
            pl.BlockSpec((units, batch), lambda: (0, 0)),
        ],
        out_specs=pl.BlockSpec((units, batch), lambda: (0, 0)),
        out_shape=jax.ShapeDtypeStruct((units, batch), jnp.float32),
    )(scale2, yT)

    return outT.T
